# Initial kernel scaffold; baseline (speedup 1.0000x reference)
import jax
import jax.numpy as jnp
from jax.experimental import pallas as pl


def kernel(surface_pos, init_ligand_pos, batch_surface, batch_ligand, time, W_s, b_s, W_t1, b_t1, W_t2, b_t2, W_csl, b_csl, W_gate, b_gate, W_hb, W_gcn, b_gcn, W_pos, b_pos):
    raise NotImplementedError("write your pallas kernel here")


# speedup vs baseline: 4.5330x; 4.5330x over previous
import functools
import jax
import jax.numpy as jnp
import numpy as np
from jax import lax
from jax.experimental import pallas as pl
from jax.experimental.pallas import tpu as pltpu
from jax.experimental.pallas import tpu_sc as plsc

HID = 128
TED = 128
NLAYERS = 4
K = 30
NS = 8000
NL = 2000
N = NS + NL
NPAD = 10240
BR = 128
NBLK = NPAD // BR

SC_CORES = 2
SC_SUBCORES = 16
NW = SC_CORES * SC_SUBCORES
NODES_PER_W = NPAD // NW
CHUNK = 8
NCHUNKS = NODES_PER_W // CHUNK
DEG = K + 1
ROWS_PER_CHUNK = CHUNK * DEG


def _sinusoidal(x, dim):
    half = dim // 2
    freq = jnp.exp(jnp.arange(half, dtype=jnp.float32) * (-np.log(10000.0) / (half - 1)))
    emb = x[:, None] * freq[None, :]
    return jnp.concatenate([jnp.sin(emb), jnp.cos(emb)], axis=-1)


def _knn_body(p_blk_ref, pt_ref, idx_ref, d2_ref):
    i = pl.program_id(0)
    p_blk = p_blk_ref[...]
    pt = pt_ref[...]
    sq_r = jnp.sum(p_blk * p_blk, axis=1)
    sq_c = jnp.sum(pt * pt, axis=0)
    dot = jnp.dot(p_blk, pt, preferred_element_type=jnp.float32)
    row_id = i * BR + lax.broadcasted_iota(jnp.int32, (BR, NPAD), 0)
    col_id = lax.broadcasted_iota(jnp.int32, (BR, NPAD), 1)
    d2 = sq_r[:, None] + sq_c[None, :] - 2.0 * dot
    bad = (col_id == row_id) | (col_id >= N)
    d2_ref[...] = jnp.where(bad, jnp.inf, d2)
    for k in range(K):
        d2 = d2_ref[...]
        v = jnp.min(d2, axis=1)
        hit = d2 == v[:, None]
        a = jnp.min(jnp.where(hit, col_id, NPAD), axis=1)
        idx_ref[:, k : k + 1] = a[:, None]
        d2_ref[...] = jnp.where(col_id == a[:, None], jnp.inf, d2)


def _knn_call(points_pad):
    return pl.pallas_call(
        _knn_body,
        grid=(NBLK,),
        in_specs=[
            pl.BlockSpec((BR, 128), lambda i: (i, 0)),
            pl.BlockSpec((128, NPAD), lambda i: (0, 0)),
        ],
        out_specs=pl.BlockSpec((BR, 128), lambda i: (i, 0)),
        out_shape=jax.ShapeDtypeStruct((NPAD, 128), jnp.int32),
        scratch_shapes=[pltpu.VMEM((BR, NPAD), jnp.float32)],
    )(points_pad, points_pad.T)[:, :K]


def _mm_body(x_ref, w_ref, o_ref):
    o_ref[...] = jnp.dot(x_ref[...], w_ref[...], preferred_element_type=jnp.float32)


def _matmul(x, w):
    mb = 1024
    return pl.pallas_call(
        _mm_body,
        grid=(NPAD // mb,),
        in_specs=[
            pl.BlockSpec((mb, 128), lambda i: (i, 0)),
            pl.BlockSpec((128, 128), lambda i: (0, 0)),
        ],
        out_specs=pl.BlockSpec((mb, 128), lambda i: (i, 0)),
        out_shape=jax.ShapeDtypeStruct((NPAD, 128), jnp.float32),
    )(x, w)


def _proj_body(x_ref, w_ref, b_ref, o_ref):
    o_ref[...] = (
        jnp.dot(x_ref[...], w_ref[...], preferred_element_type=jnp.float32)
        + b_ref[...]
    )


def _proj(x, w_pad, b_pad):
    return pl.pallas_call(
        _proj_body,
        in_specs=[
            pl.BlockSpec((2048, 128), lambda: (0, 0)),
            pl.BlockSpec((128, 128), lambda: (0, 0)),
            pl.BlockSpec((1, 128), lambda: (0, 0)),
        ],
        out_specs=pl.BlockSpec((2048, 128), lambda: (0, 0)),
        out_shape=jax.ShapeDtypeStruct((2048, 128), jnp.float32),
    )(x, w_pad, b_pad)


def _gather_body(m_hbm, idx_hbm, b_hbm, out_hbm, idx_v, rows_v, acc_v, b_v, sem):
    wid = lax.axis_index("s") * SC_CORES + lax.axis_index("c")
    pltpu.sync_copy(b_hbm, b_v)

    def chunk(c, carry):
        base = wid * NODES_PER_W + c * CHUNK
        pltpu.sync_copy(idx_hbm.at[pl.ds(base * DEG, ROWS_PER_CHUNK)], idx_v)
        pltpu.async_copy(m_hbm.at[idx_v], rows_v, sem).wait()
        for j in range(CHUNK):
            for lc in range(8):
                def rbody(r, a):
                    return a + rows_v[j * DEG + r, pl.ds(lc * 16, 16)]
                a = lax.fori_loop(0, DEG, rbody, jnp.zeros((16,), jnp.float32))
                acc_v[j, pl.ds(lc * 16, 16)] = a * (1.0 / DEG) + b_v[pl.ds(lc * 16, 16)]
        pltpu.sync_copy(acc_v, out_hbm.at[pl.ds(base, CHUNK)])
        return carry

    lax.fori_loop(0, NCHUNKS, chunk, 0)


def _make_gather_kernel():
    return pl.kernel(
        _gather_body,
        mesh=plsc.VectorSubcoreMesh(core_axis_name="c", subcore_axis_name="s"),
        out_type=jax.ShapeDtypeStruct((NPAD, HID), jnp.float32),
        scratch_types=[
            pltpu.VMEM((ROWS_PER_CHUNK,), jnp.int32),
            pltpu.VMEM((ROWS_PER_CHUNK, HID), jnp.float32),
            pltpu.VMEM((CHUNK, HID), jnp.float32),
            pltpu.VMEM((HID,), jnp.float32),
            pltpu.SemaphoreType.DMA,
        ],
    )


def kernel(surface_pos, init_ligand_pos, batch_surface, batch_ligand, time,
           W_s, b_s, W_t1, b_t1, W_t2, b_t2, W_csl, b_csl, W_gate, b_gate,
           W_hb, W_gcn, b_gcn, W_pos, b_pos):
    h_surface = surface_pos @ W_s + b_s
    t = _sinusoidal(jnp.squeeze(time, -1), TED)
    h_time = jax.nn.gelu(t @ W_t1 + b_t1) @ W_t2 + b_t2
    h_lig = (init_ligand_pos @ W_csl + b_csl) * jax.nn.sigmoid(
        h_time @ W_gate + b_gate) + h_time @ W_hb

    pos = jnp.concatenate([surface_pos, init_ligand_pos], axis=0)
    points_pad = jnp.zeros((NPAD, 128), jnp.float32).at[:N, :3].set(pos)

    idx = _knn_call(points_pad)[:N]

    self_col = jnp.arange(N, dtype=jnp.int32)[:, None]
    idx_full = jnp.concatenate([idx, self_col], axis=1)
    idx_flat = jnp.zeros((NPAD * DEG,), jnp.int32).at[: N * DEG].set(
        idx_full.reshape(-1))

    h = jnp.zeros((NPAD, HID), jnp.float32)
    h = h.at[:N].set(jnp.concatenate([h_surface, h_lig], axis=0))
    gather_kernel = _make_gather_kernel()
    for i in range(NLAYERS):
        m = _matmul(h, W_gcn[i])
        h = gather_kernel(m, idx_flat, b_gcn[i])

    x = jnp.zeros((2048, 128), jnp.float32).at[:NL].set(h[NS:N])
    w_pad = jnp.zeros((128, 128), jnp.float32).at[:, :3].set(W_pos)
    b_pad = jnp.zeros((1, 128), jnp.float32).at[0, :3].set(b_pos)
    y = _proj(x, w_pad, b_pad)
    return y[:NL, :3]


# speedup vs baseline: 5.3845x; 1.1878x over previous
import functools
import jax
import jax.numpy as jnp
import numpy as np
from jax import lax
from jax.experimental import pallas as pl
from jax.experimental.pallas import tpu as pltpu
from jax.experimental.pallas import tpu_sc as plsc

HID = 128
TED = 128
NLAYERS = 4
K = 30
NS = 8000
NL = 2000
N = NS + NL
NPAD = 10240
BR = 128
NBLK = NPAD // BR

SC_CORES = 2
SC_SUBCORES = 16
NW = SC_CORES * SC_SUBCORES
NODES_PER_W = NPAD // NW
CHUNK = 8
NCHUNKS = NODES_PER_W // CHUNK
DEG = K + 1
ROWS_PER_CHUNK = CHUNK * DEG


def _sinusoidal(x, dim):
    half = dim // 2
    freq = jnp.exp(jnp.arange(half, dtype=jnp.float32) * (-np.log(10000.0) / (half - 1)))
    emb = x[:, None] * freq[None, :]
    return jnp.concatenate([jnp.sin(emb), jnp.cos(emb)], axis=-1)


def _knn_body(p_blk_ref, pt_ref, idx_ref, d2_ref):
    i = pl.program_id(0)
    p_blk = p_blk_ref[...]
    pt = pt_ref[...]
    sq_r = jnp.sum(p_blk * p_blk, axis=1)
    sq_c = jnp.sum(pt * pt, axis=0)
    dot = jnp.dot(p_blk, pt, preferred_element_type=jnp.float32)
    row_id = i * BR + lax.broadcasted_iota(jnp.int32, (BR, NPAD), 0)
    col_id = lax.broadcasted_iota(jnp.int32, (BR, NPAD), 1)
    d2 = sq_r[:, None] + sq_c[None, :] - 2.0 * dot
    bad = (col_id == row_id) | (col_id >= N)
    d2_ref[...] = jnp.where(bad, jnp.inf, d2)
    for k in range(K):
        d2 = d2_ref[...]
        a = jnp.argmin(d2, axis=1).astype(jnp.int32)
        idx_ref[:, k : k + 1] = a[:, None]
        d2_ref[...] = jnp.where(col_id == a[:, None], jnp.inf, d2)


def _knn_call(points_pad):
    return pl.pallas_call(
        _knn_body,
        grid=(NBLK,),
        in_specs=[
            pl.BlockSpec((BR, 128), lambda i: (i, 0)),
            pl.BlockSpec((128, NPAD), lambda i: (0, 0)),
        ],
        out_specs=pl.BlockSpec((BR, 128), lambda i: (i, 0)),
        out_shape=jax.ShapeDtypeStruct((NPAD, 128), jnp.int32),
        scratch_shapes=[pltpu.VMEM((BR, NPAD), jnp.float32)],
    )(points_pad, points_pad.T)[:, :K]


def _mm_body(x_ref, w_ref, o_ref):
    o_ref[...] = jnp.dot(x_ref[...], w_ref[...], preferred_element_type=jnp.float32)


def _matmul(x, w):
    mb = 1024
    return pl.pallas_call(
        _mm_body,
        grid=(NPAD // mb,),
        in_specs=[
            pl.BlockSpec((mb, 128), lambda i: (i, 0)),
            pl.BlockSpec((128, 128), lambda i: (0, 0)),
        ],
        out_specs=pl.BlockSpec((mb, 128), lambda i: (i, 0)),
        out_shape=jax.ShapeDtypeStruct((NPAD, 128), jnp.float32),
    )(x, w)


def _proj_body(x_ref, w_ref, b_ref, o_ref):
    o_ref[...] = (
        jnp.dot(x_ref[...], w_ref[...], preferred_element_type=jnp.float32)
        + b_ref[...]
    )


def _proj(x, w_pad, b_pad):
    return pl.pallas_call(
        _proj_body,
        in_specs=[
            pl.BlockSpec((2048, 128), lambda: (0, 0)),
            pl.BlockSpec((128, 128), lambda: (0, 0)),
            pl.BlockSpec((1, 128), lambda: (0, 0)),
        ],
        out_specs=pl.BlockSpec((2048, 128), lambda: (0, 0)),
        out_shape=jax.ShapeDtypeStruct((2048, 128), jnp.float32),
    )(x, w_pad, b_pad)


def _gather_body(m_hbm, idx_hbm, b_hbm, out_hbm, idx0, idx1, rows0, rows1,
                 acc_v, b_v, sem0, sem1):
    wid = lax.axis_index("s") * SC_CORES + lax.axis_index("c")
    pltpu.sync_copy(b_hbm, b_v)
    wbase = wid * NODES_PER_W

    def fetch_idx(idx_v, c):
        pltpu.sync_copy(
            idx_hbm.at[pl.ds((wbase + c * CHUNK) * DEG, ROWS_PER_CHUNK)], idx_v)

    def accum(rows_v, c):
        base = wbase + c * CHUNK
        for j in range(CHUNK):
            for lc in range(8):
                def rbody(r, a):
                    return a + rows_v[j * DEG + r, pl.ds(lc * 16, 16)]
                a = lax.fori_loop(0, DEG, rbody, jnp.zeros((16,), jnp.float32))
                acc_v[j, pl.ds(lc * 16, 16)] = a * (1.0 / DEG) + b_v[pl.ds(lc * 16, 16)]
        pltpu.sync_copy(acc_v, out_hbm.at[pl.ds(base, CHUNK)])

    fetch_idx(idx0, 0)
    pltpu.make_async_copy(m_hbm.at[idx0], rows0, sem0).start()

    def step(t, carry):
        c0 = 2 * t
        c1 = c0 + 1
        fetch_idx(idx1, c1)
        pltpu.make_async_copy(m_hbm.at[idx1], rows1, sem1).start()
        pltpu.make_async_copy(m_hbm.at[idx0], rows0, sem0).wait()
        accum(rows0, c0)

        @pl.when(t < NCHUNKS // 2 - 1)
        def _():
            fetch_idx(idx0, c0 + 2)
            pltpu.make_async_copy(m_hbm.at[idx0], rows0, sem0).start()

        pltpu.make_async_copy(m_hbm.at[idx1], rows1, sem1).wait()
        accum(rows1, c1)
        return carry

    lax.fori_loop(0, NCHUNKS // 2, step, 0)


def _make_gather_kernel():
    return pl.kernel(
        _gather_body,
        mesh=plsc.VectorSubcoreMesh(core_axis_name="c", subcore_axis_name="s"),
        out_type=jax.ShapeDtypeStruct((NPAD, HID), jnp.float32),
        scratch_types=[
            pltpu.VMEM((ROWS_PER_CHUNK,), jnp.int32),
            pltpu.VMEM((ROWS_PER_CHUNK,), jnp.int32),
            pltpu.VMEM((ROWS_PER_CHUNK, HID), jnp.float32),
            pltpu.VMEM((ROWS_PER_CHUNK, HID), jnp.float32),
            pltpu.VMEM((CHUNK, HID), jnp.float32),
            pltpu.VMEM((HID,), jnp.float32),
            pltpu.SemaphoreType.DMA,
            pltpu.SemaphoreType.DMA,
        ],
    )


def kernel(surface_pos, init_ligand_pos, batch_surface, batch_ligand, time,
           W_s, b_s, W_t1, b_t1, W_t2, b_t2, W_csl, b_csl, W_gate, b_gate,
           W_hb, W_gcn, b_gcn, W_pos, b_pos):
    h_surface = surface_pos @ W_s + b_s
    t = _sinusoidal(jnp.squeeze(time, -1), TED)
    h_time = jax.nn.gelu(t @ W_t1 + b_t1) @ W_t2 + b_t2
    h_lig = (init_ligand_pos @ W_csl + b_csl) * jax.nn.sigmoid(
        h_time @ W_gate + b_gate) + h_time @ W_hb

    pos = jnp.concatenate([surface_pos, init_ligand_pos], axis=0)
    points_pad = jnp.zeros((NPAD, 128), jnp.float32).at[:N, :3].set(pos)

    idx = _knn_call(points_pad)[:N]

    self_col = jnp.arange(N, dtype=jnp.int32)[:, None]
    idx_full = jnp.concatenate([idx, self_col], axis=1)
    idx_flat = jnp.zeros((NPAD * DEG,), jnp.int32).at[: N * DEG].set(
        idx_full.reshape(-1))

    h = jnp.zeros((NPAD, HID), jnp.float32)
    h = h.at[:N].set(jnp.concatenate([h_surface, h_lig], axis=0))
    gather_kernel = _make_gather_kernel()
    for i in range(NLAYERS):
        m = _matmul(h, W_gcn[i])
        h = gather_kernel(m, idx_flat, b_gcn[i])

    x = jnp.zeros((2048, 128), jnp.float32).at[:NL].set(h[NS:N])
    w_pad = jnp.zeros((128, 128), jnp.float32).at[:, :3].set(W_pos)
    b_pad = jnp.zeros((1, 128), jnp.float32).at[0, :3].set(b_pos)
    y = _proj(x, w_pad, b_pad)
    return y[:NL, :3]


# speedup vs baseline: 5.4830x; 1.0183x over previous
import functools
import jax
import jax.numpy as jnp
import numpy as np
from jax import lax
from jax.experimental import pallas as pl
from jax.experimental.pallas import tpu as pltpu
from jax.experimental.pallas import tpu_sc as plsc

HID = 128
TED = 128
NLAYERS = 4
K = 30
NS = 8000
NL = 2000
N = NS + NL
NPAD = 10240
BR = 128
NBLK = NPAD // BR

SC_CORES = 2
SC_SUBCORES = 16
NW = SC_CORES * SC_SUBCORES
NODES_PER_W = NPAD // NW
CHUNK = 8
NCHUNKS = NODES_PER_W // CHUNK
DEG = K + 1
ROWS_PER_CHUNK = CHUNK * DEG


def _sinusoidal(x, dim):
    half = dim // 2
    freq = jnp.exp(jnp.arange(half, dtype=jnp.float32) * (-np.log(10000.0) / (half - 1)))
    emb = x[:, None] * freq[None, :]
    return jnp.concatenate([jnp.sin(emb), jnp.cos(emb)], axis=-1)


def _knn_body(p_blk_ref, pt_ref, idx_ref, d2_ref):
    i = pl.program_id(0)
    p_blk = p_blk_ref[...]
    pt = pt_ref[...]
    sq_r = jnp.sum(p_blk * p_blk, axis=1)
    sq_c = jnp.sum(pt * pt, axis=0)
    dot = jnp.dot(p_blk, pt, preferred_element_type=jnp.float32)
    row_id = i * BR + lax.broadcasted_iota(jnp.int32, (BR, NPAD), 0)
    col_id = lax.broadcasted_iota(jnp.int32, (BR, NPAD), 1)
    d2 = sq_r[:, None] + sq_c[None, :] - 2.0 * dot
    bad = (col_id == row_id) | (col_id >= N)
    d2_ref[...] = jnp.where(bad, jnp.inf, d2)
    for k in range(K):
        d2 = d2_ref[...]
        a = jnp.argmin(d2, axis=1).astype(jnp.int32)
        idx_ref[:, k : k + 1] = a[:, None]
        d2_ref[...] = jnp.where(col_id == a[:, None], jnp.inf, d2)


def _knn_call(points_pad):
    return pl.pallas_call(
        _knn_body,
        grid=(NBLK,),
        in_specs=[
            pl.BlockSpec((BR, 128), lambda i: (i, 0)),
            pl.BlockSpec((128, NPAD), lambda i: (0, 0)),
        ],
        out_specs=pl.BlockSpec((BR, 128), lambda i: (i, 0)),
        out_shape=jax.ShapeDtypeStruct((NPAD, 128), jnp.int32),
        scratch_shapes=[pltpu.VMEM((BR, NPAD), jnp.float32)],
    )(points_pad, points_pad.T)[:, :K]


def _mm_body(x_ref, w_ref, o_ref):
    o_ref[...] = jnp.dot(x_ref[...], w_ref[...], preferred_element_type=jnp.float32)


def _matmul(x, w):
    mb = 1024
    return pl.pallas_call(
        _mm_body,
        grid=(NPAD // mb,),
        in_specs=[
            pl.BlockSpec((mb, 128), lambda i: (i, 0)),
            pl.BlockSpec((128, 128), lambda i: (0, 0)),
        ],
        out_specs=pl.BlockSpec((mb, 128), lambda i: (i, 0)),
        out_shape=jax.ShapeDtypeStruct((NPAD, 128), jnp.float32),
    )(x, w)


def _proj_body(x_ref, w_ref, b_ref, o_ref):
    o_ref[...] = (
        jnp.dot(x_ref[...], w_ref[...], preferred_element_type=jnp.float32)
        + b_ref[...]
    )


def _proj(x, w_pad, b_pad):
    return pl.pallas_call(
        _proj_body,
        in_specs=[
            pl.BlockSpec((2048, 128), lambda: (0, 0)),
            pl.BlockSpec((128, 128), lambda: (0, 0)),
            pl.BlockSpec((1, 128), lambda: (0, 0)),
        ],
        out_specs=pl.BlockSpec((2048, 128), lambda: (0, 0)),
        out_shape=jax.ShapeDtypeStruct((2048, 128), jnp.float32),
    )(x, w_pad, b_pad)


def _gather_body(m_hbm, idx_hbm, b_hbm, out_hbm, idx0, idx1, rows0, rows1,
                 acc_v, b_v, sem0, sem1):
    wid = lax.axis_index("s") * SC_CORES + lax.axis_index("c")
    pltpu.sync_copy(b_hbm, b_v)
    wbase = wid * NODES_PER_W

    def fetch_idx(idx_v, c):
        pltpu.sync_copy(
            idx_hbm.at[pl.ds((wbase + c * CHUNK) * DEG, ROWS_PER_CHUNK)], idx_v)

    def accum(rows_v, c):
        base = wbase + c * CHUNK
        for j in range(CHUNK):
            def rbody(r, accs):
                row = j * DEG + r
                return tuple(
                    accs[lc] + rows_v[row, pl.ds(lc * 16, 16)] for lc in range(8))
            zero = jnp.zeros((16,), jnp.float32)
            accs = lax.fori_loop(0, DEG, rbody, (zero,) * 8)
            for lc in range(8):
                acc_v[j, pl.ds(lc * 16, 16)] = (
                    accs[lc] * (1.0 / DEG) + b_v[pl.ds(lc * 16, 16)])
        pltpu.sync_copy(acc_v, out_hbm.at[pl.ds(base, CHUNK)])

    fetch_idx(idx0, 0)
    pltpu.make_async_copy(m_hbm.at[idx0], rows0, sem0).start()

    def step(t, carry):
        c0 = 2 * t
        c1 = c0 + 1
        fetch_idx(idx1, c1)
        pltpu.make_async_copy(m_hbm.at[idx1], rows1, sem1).start()
        pltpu.make_async_copy(m_hbm.at[idx0], rows0, sem0).wait()
        accum(rows0, c0)

        @pl.when(t < NCHUNKS // 2 - 1)
        def _():
            fetch_idx(idx0, c0 + 2)
            pltpu.make_async_copy(m_hbm.at[idx0], rows0, sem0).start()

        pltpu.make_async_copy(m_hbm.at[idx1], rows1, sem1).wait()
        accum(rows1, c1)
        return carry

    lax.fori_loop(0, NCHUNKS // 2, step, 0)


def _make_gather_kernel():
    return pl.kernel(
        _gather_body,
        mesh=plsc.VectorSubcoreMesh(core_axis_name="c", subcore_axis_name="s"),
        out_type=jax.ShapeDtypeStruct((NPAD, HID), jnp.float32),
        scratch_types=[
            pltpu.VMEM((ROWS_PER_CHUNK,), jnp.int32),
            pltpu.VMEM((ROWS_PER_CHUNK,), jnp.int32),
            pltpu.VMEM((ROWS_PER_CHUNK, HID), jnp.float32),
            pltpu.VMEM((ROWS_PER_CHUNK, HID), jnp.float32),
            pltpu.VMEM((CHUNK, HID), jnp.float32),
            pltpu.VMEM((HID,), jnp.float32),
            pltpu.SemaphoreType.DMA,
            pltpu.SemaphoreType.DMA,
        ],
    )


def kernel(surface_pos, init_ligand_pos, batch_surface, batch_ligand, time,
           W_s, b_s, W_t1, b_t1, W_t2, b_t2, W_csl, b_csl, W_gate, b_gate,
           W_hb, W_gcn, b_gcn, W_pos, b_pos):
    h_surface = surface_pos @ W_s + b_s
    t = _sinusoidal(jnp.squeeze(time, -1), TED)
    h_time = jax.nn.gelu(t @ W_t1 + b_t1) @ W_t2 + b_t2
    h_lig = (init_ligand_pos @ W_csl + b_csl) * jax.nn.sigmoid(
        h_time @ W_gate + b_gate) + h_time @ W_hb

    pos = jnp.concatenate([surface_pos, init_ligand_pos], axis=0)
    points_pad = jnp.zeros((NPAD, 128), jnp.float32).at[:N, :3].set(pos)

    idx = _knn_call(points_pad)[:N]

    self_col = jnp.arange(N, dtype=jnp.int32)[:, None]
    idx_full = jnp.concatenate([idx, self_col], axis=1)
    idx_flat = jnp.zeros((NPAD * DEG,), jnp.int32).at[: N * DEG].set(
        idx_full.reshape(-1))

    h = jnp.zeros((NPAD, HID), jnp.float32)
    h = h.at[:N].set(jnp.concatenate([h_surface, h_lig], axis=0))
    gather_kernel = _make_gather_kernel()
    for i in range(NLAYERS):
        m = _matmul(h, W_gcn[i])
        h = gather_kernel(m, idx_flat, b_gcn[i])

    x = jnp.zeros((2048, 128), jnp.float32).at[:NL].set(h[NS:N])
    w_pad = jnp.zeros((128, 128), jnp.float32).at[:, :3].set(W_pos)
    b_pad = jnp.zeros((1, 128), jnp.float32).at[0, :3].set(b_pos)
    y = _proj(x, w_pad, b_pad)
    return y[:NL, :3]


# speedup vs baseline: 6.2267x; 1.1356x over previous
import functools
import jax
import jax.numpy as jnp
import numpy as np
from jax import lax
from jax.experimental import pallas as pl
from jax.experimental.pallas import tpu as pltpu
from jax.experimental.pallas import tpu_sc as plsc

HID = 128
TED = 128
NLAYERS = 4
K = 30
NS = 8000
NL = 2000
N = NS + NL
NPAD = 10240
BR = 128
NBLK = NPAD // BR

SC_CORES = 2
SC_SUBCORES = 16
NW = SC_CORES * SC_SUBCORES
NODES_PER_W = NPAD // NW
CHUNK = 8
NCHUNKS = NODES_PER_W // CHUNK
DEG = K + 1
ROWS_PER_CHUNK = CHUNK * DEG


def _sinusoidal(x, dim):
    half = dim // 2
    freq = jnp.exp(jnp.arange(half, dtype=jnp.float32) * (-np.log(10000.0) / (half - 1)))
    emb = x[:, None] * freq[None, :]
    return jnp.concatenate([jnp.sin(emb), jnp.cos(emb)], axis=-1)


def _knn_body(p_blk_ref, pt_ref, idx_ref, d2_ref):
    i = pl.program_id(0)
    p_blk = p_blk_ref[...]
    pt = pt_ref[...]
    sq_r = jnp.sum(p_blk * p_blk, axis=1)
    sq_c = jnp.sum(pt * pt, axis=0)
    dot = jnp.dot(p_blk, pt, preferred_element_type=jnp.float32)
    row_id = i * BR + lax.broadcasted_iota(jnp.int32, (BR, NPAD), 0)
    col_id = lax.broadcasted_iota(jnp.int32, (BR, NPAD), 1)
    d2 = sq_r[:, None] + sq_c[None, :] - 2.0 * dot
    bad = (col_id == row_id) | (col_id >= N)
    d2_ref[...] = jnp.where(bad, jnp.inf, d2)
    for k in range(K):
        d2 = d2_ref[...]
        a = jnp.argmin(d2, axis=1).astype(jnp.int32)
        idx_ref[:, k : k + 1] = a[:, None]
        d2_ref[...] = jnp.where(col_id == a[:, None], jnp.inf, d2)


def _knn_call(points_pad):
    return pl.pallas_call(
        _knn_body,
        grid=(NBLK,),
        in_specs=[
            pl.BlockSpec((BR, 128), lambda i: (i, 0)),
            pl.BlockSpec((128, NPAD), lambda i: (0, 0)),
        ],
        out_specs=pl.BlockSpec((BR, 128), lambda i: (i, 0)),
        out_shape=jax.ShapeDtypeStruct((NPAD, 128), jnp.int32),
        scratch_shapes=[pltpu.VMEM((BR, NPAD), jnp.float32)],
    )(points_pad, points_pad.T)[:, :K]


def _mm_body(x_ref, w_ref, o_ref):
    o_ref[...] = jnp.dot(x_ref[...], w_ref[...], preferred_element_type=jnp.float32)


def _matmul(x, w):
    mb = 1024
    return pl.pallas_call(
        _mm_body,
        grid=(NPAD // mb,),
        in_specs=[
            pl.BlockSpec((mb, 128), lambda i: (i, 0)),
            pl.BlockSpec((128, 128), lambda i: (0, 0)),
        ],
        out_specs=pl.BlockSpec((mb, 128), lambda i: (i, 0)),
        out_shape=jax.ShapeDtypeStruct((NPAD, 128), jnp.float32),
    )(x, w)


def _proj_body(x_ref, w_ref, b_ref, o_ref):
    o_ref[...] = (
        jnp.dot(x_ref[...], w_ref[...], preferred_element_type=jnp.float32)
        + b_ref[...]
    )


def _proj(x, w_pad, b_pad):
    return pl.pallas_call(
        _proj_body,
        in_specs=[
            pl.BlockSpec((2048, 128), lambda: (0, 0)),
            pl.BlockSpec((128, 128), lambda: (0, 0)),
            pl.BlockSpec((1, 128), lambda: (0, 0)),
        ],
        out_specs=pl.BlockSpec((2048, 128), lambda: (0, 0)),
        out_shape=jax.ShapeDtypeStruct((2048, 128), jnp.float32),
    )(x, w_pad, b_pad)


def _gather_body(nodes_per_w, nchunks, m_hbm, idx_hbm, b_hbm, out_hbm, idx0,
                 idx1, rows0, rows1, acc_v, b_v, sem0, sem1):
    wid = lax.axis_index("s") * SC_CORES + lax.axis_index("c")
    pltpu.sync_copy(b_hbm, b_v)
    wbase = wid * nodes_per_w

    def fetch_idx(idx_v, c):
        pltpu.sync_copy(
            idx_hbm.at[pl.ds((wbase + c * CHUNK) * DEG, ROWS_PER_CHUNK)], idx_v)

    def accum(rows_v, c):
        base = wbase + c * CHUNK
        for j in range(CHUNK):
            def rbody(r, accs):
                row = j * DEG + r
                return tuple(
                    accs[lc] + rows_v[row, pl.ds(lc * 16, 16)] for lc in range(8))
            zero = jnp.zeros((16,), jnp.float32)
            accs = lax.fori_loop(0, DEG, rbody, (zero,) * 8)
            for lc in range(8):
                acc_v[j, pl.ds(lc * 16, 16)] = (
                    accs[lc] * (1.0 / DEG) + b_v[pl.ds(lc * 16, 16)])
        pltpu.sync_copy(acc_v, out_hbm.at[pl.ds(base, CHUNK)])

    fetch_idx(idx0, 0)
    pltpu.make_async_copy(m_hbm.at[idx0], rows0, sem0).start()

    def step(t, carry):
        c0 = 2 * t
        c1 = c0 + 1
        fetch_idx(idx1, c1)
        pltpu.make_async_copy(m_hbm.at[idx1], rows1, sem1).start()
        pltpu.make_async_copy(m_hbm.at[idx0], rows0, sem0).wait()
        accum(rows0, c0)

        @pl.when(t < nchunks // 2 - 1)
        def _():
            fetch_idx(idx0, c0 + 2)
            pltpu.make_async_copy(m_hbm.at[idx0], rows0, sem0).start()

        pltpu.make_async_copy(m_hbm.at[idx1], rows1, sem1).wait()
        accum(rows1, c1)
        return carry

    lax.fori_loop(0, nchunks // 2, step, 0)


def _make_gather_kernel(n_out):
    nodes_per_w = n_out // NW
    nchunks = nodes_per_w // CHUNK
    return pl.kernel(
        functools.partial(_gather_body, nodes_per_w, nchunks),
        mesh=plsc.VectorSubcoreMesh(core_axis_name="c", subcore_axis_name="s"),
        out_type=jax.ShapeDtypeStruct((n_out, HID), jnp.float32),
        scratch_types=[
            pltpu.VMEM((ROWS_PER_CHUNK,), jnp.int32),
            pltpu.VMEM((ROWS_PER_CHUNK,), jnp.int32),
            pltpu.VMEM((ROWS_PER_CHUNK, HID), jnp.float32),
            pltpu.VMEM((ROWS_PER_CHUNK, HID), jnp.float32),
            pltpu.VMEM((CHUNK, HID), jnp.float32),
            pltpu.VMEM((HID,), jnp.float32),
            pltpu.SemaphoreType.DMA,
            pltpu.SemaphoreType.DMA,
        ],
    )


def kernel(surface_pos, init_ligand_pos, batch_surface, batch_ligand, time,
           W_s, b_s, W_t1, b_t1, W_t2, b_t2, W_csl, b_csl, W_gate, b_gate,
           W_hb, W_gcn, b_gcn, W_pos, b_pos):
    h_surface = surface_pos @ W_s + b_s
    t = _sinusoidal(jnp.squeeze(time, -1), TED)
    h_time = jax.nn.gelu(t @ W_t1 + b_t1) @ W_t2 + b_t2
    h_lig = (init_ligand_pos @ W_csl + b_csl) * jax.nn.sigmoid(
        h_time @ W_gate + b_gate) + h_time @ W_hb

    pos = jnp.concatenate([surface_pos, init_ligand_pos], axis=0)
    points_pad = jnp.zeros((NPAD, 128), jnp.float32).at[:N, :3].set(pos)

    idx = _knn_call(points_pad)[:N]

    self_col = jnp.arange(N, dtype=jnp.int32)[:, None]
    idx_full = jnp.concatenate([idx, self_col], axis=1)
    idx_flat = jnp.zeros((NPAD * DEG,), jnp.int32).at[: N * DEG].set(
        idx_full.reshape(-1))
    idx_lig = jnp.zeros((2048 * DEG,), jnp.int32).at[: NL * DEG].set(
        idx_full[NS:].reshape(-1))

    h = jnp.zeros((NPAD, HID), jnp.float32)
    h = h.at[:N].set(jnp.concatenate([h_surface, h_lig], axis=0))
    gather_all = _make_gather_kernel(NPAD)
    gather_lig = _make_gather_kernel(2048)
    for i in range(NLAYERS - 1):
        m = _matmul(h, W_gcn[i])
        h = gather_all(m, idx_flat, b_gcn[i])
    m = _matmul(h, W_gcn[NLAYERS - 1])
    x = gather_lig(m, idx_lig, b_gcn[NLAYERS - 1])

    w_pad = jnp.zeros((128, 128), jnp.float32).at[:, :3].set(W_pos)
    b_pad = jnp.zeros((1, 128), jnp.float32).at[0, :3].set(b_pos)
    y = _proj(x, w_pad, b_pad)
    return y[:NL, :3]
